# Initial kernel scaffold; baseline (speedup 1.0000x reference)
import jax
import jax.numpy as jnp
from jax.experimental import pallas as pl


def kernel(x, edge_index, edge_attr, W1, b1, W2, b2, W3, b3, root, ncb, L1, bL1, L2, bL2):
    raise NotImplementedError("write your pallas kernel here")


# speedup vs baseline: 3.5680x; 3.5680x over previous
import functools

import jax
import jax.numpy as jnp
from jax import lax
from jax.experimental import pallas as pl
from jax.experimental.pallas import tpu as pltpu
from jax.experimental.pallas import tpu_sc as plsc

N = 10000
E = 160000
IN = 32
OUT = 32
HID = 32
ED = 16
EPS = 1e-5

NW = 32
CH = 128
E_PAD = 163840
CPW = E_PAD // (NW * CH)
B = 2048
GRID = E_PAD // B

_f32 = jnp.float32


def _stats_root_body(x_ref, root_ref, ncb_ref, stats_ref, r_ref):
    x = x_ref[...]
    mean = jnp.mean(x, axis=0, keepdims=True)
    xc = x - mean
    var = jnp.mean(xc * xc, axis=0, keepdims=True)
    istd = lax.rsqrt(var + EPS)
    xn = xc * istd
    r_ref[...] = jnp.dot(xn, root_ref[...], preferred_element_type=_f32) + ncb_ref[...]
    stats_ref[...] = jnp.concatenate(
        [mean, istd, jnp.zeros((6, IN), _f32)], axis=0)


def _msg_body(eaT_ref, xg_ref, stats_ref, w1t_ref, b1_ref, w2t_ref, b2_ref,
              w3t_ref, b3_ref, out_ref):
    b = pl.program_id(0)
    eaT = eaT_ref[...]
    h1 = jnp.maximum(jnp.dot(w1t_ref[...], eaT, preferred_element_type=_f32)
                     + b1_ref[...], 0.0)
    h2 = jnp.maximum(jnp.dot(w2t_ref[...], h1, preferred_element_type=_f32)
                     + b2_ref[...], 0.0)
    wT = jnp.maximum(jnp.dot(w3t_ref[...], h2, preferred_element_type=_f32)
                     + b3_ref[...], 0.0)
    stats = stats_ref[...]
    xg = xg_ref[...]
    xgn = (xg - stats[0:1, :]) * stats[1:2, :]
    xgnT = xgn.T
    acc = jnp.zeros((HID, B), _f32)
    for i in range(IN):
        row = jnp.broadcast_to(xgnT[i:i + 1, :], (HID, B))
        acc = acc + wT[HID * i:HID * (i + 1), :] * row
    col = lax.broadcasted_iota(jnp.int32, (HID, B), 1) + b * B
    acc = jnp.where(col < E, acc, 0.0)
    out_ref[...] = acc.T


def _final_body(p_ref, r_ref, l1_ref, bl1_ref, l2_ref, bl2_ref, out_ref):
    agg = p_ref[0] + p_ref[1]
    o = jnp.maximum(agg + r_ref[...], 0.0)
    h = jnp.maximum(jnp.dot(o, l1_ref[...], preferred_element_type=_f32)
                    + bl1_ref[...], 0.0)
    out_ref[...] = jnp.dot(h, l2_ref[...], preferred_element_type=_f32) + bl2_ref[...]


def _sc_gather(x, src3):
    mesh = plsc.VectorSubcoreMesh(core_axis_name="c", subcore_axis_name="s")

    @functools.partial(
        pl.kernel, mesh=mesh,
        out_type=jax.ShapeDtypeStruct((E_PAD, IN), _f32),
        scratch_types=[
            pltpu.VMEM((CPW, CH), jnp.int32),
            pltpu.VMEM((CH, IN), _f32),
            pltpu.SemaphoreType.DMA,
        ],
        compiler_params=pltpu.CompilerParams(use_tc_tiling_on_sc=False),
    )
    def k(x_hbm, src_hbm, out_hbm, idx_v, rows_v, sem):
        wid = lax.axis_index("s") * 2 + lax.axis_index("c")
        base = wid * (CPW * CH)
        pltpu.sync_copy(src_hbm.at[wid], idx_v)

        def chunk(c, _):
            pltpu.async_copy(x_hbm.at[idx_v.at[c]], rows_v, sem).wait()
            pltpu.sync_copy(rows_v, out_hbm.at[pl.ds(base + c * CH, CH)])
            return 0

        lax.fori_loop(0, CPW, chunk, 0)

    return k(x, src3)


def _sc_scatter(msg, dst3, zeros_n):
    mesh = plsc.VectorSubcoreMesh(core_axis_name="c", subcore_axis_name="s")
    stripe = N // 16

    @functools.partial(
        pl.kernel, mesh=mesh,
        out_type=jax.ShapeDtypeStruct((2, N, HID), _f32),
        scratch_types=[
            pltpu.VMEM_SHARED((N, HID), _f32),
            pltpu.VMEM((CPW, CH), jnp.int32),
            pltpu.VMEM((CH, HID), _f32),
        ],
        compiler_params=pltpu.CompilerParams(use_tc_tiling_on_sc=False),
    )
    def k(msg_hbm, dst_hbm, zeros_hbm, out_hbm, acc_sh, idx_v, m_v):
        cid = lax.axis_index("c")
        sid = lax.axis_index("s")
        wid = sid * 2 + cid
        base = wid * (CPW * CH)
        pltpu.sync_copy(zeros_hbm.at[pl.ds(sid * stripe, stripe)],
                        acc_sh.at[pl.ds(sid * stripe, stripe)])
        pltpu.sync_copy(dst_hbm.at[wid], idx_v)
        plsc.subcore_barrier()

        def chunk(c, _):
            pltpu.sync_copy(msg_hbm.at[pl.ds(base + c * CH, CH)], m_v)
            pltpu.sync_copy(m_v, acc_sh.at[idx_v.at[c]], add=True)
            return 0

        lax.fori_loop(0, CPW, chunk, 0)
        plsc.subcore_barrier()
        pltpu.sync_copy(acc_sh.at[pl.ds(sid * stripe, stripe)],
                        out_hbm.at[cid, pl.ds(sid * stripe, stripe)])

    return k(msg, dst3, zeros_n)


def kernel(x, edge_index, edge_attr, W1, b1, W2, b2, W3, b3, root, ncb,
           L1, bL1, L2, bL2):
    pad = E_PAD - E
    src3 = jnp.pad(edge_index[0], (0, pad)).reshape(NW, CPW, CH)
    dst3 = jnp.pad(edge_index[1], (0, pad)).reshape(NW, CPW, CH)
    eaT = jnp.pad(edge_attr, ((0, pad), (0, 0))).T
    zeros_n = jnp.zeros((N, HID), _f32)

    stats, r = pl.pallas_call(
        _stats_root_body,
        out_shape=[jax.ShapeDtypeStruct((8, IN), _f32),
                   jax.ShapeDtypeStruct((N, HID), _f32)],
    )(x, root, ncb.reshape(1, HID))

    xg = _sc_gather(x, src3)

    msg = pl.pallas_call(
        _msg_body,
        grid=(GRID,),
        in_specs=[
            pl.BlockSpec((ED, B), lambda b: (0, b)),
            pl.BlockSpec((B, IN), lambda b: (b, 0)),
            pl.BlockSpec((8, IN), lambda b: (0, 0)),
            pl.BlockSpec((HID, ED), lambda b: (0, 0)),
            pl.BlockSpec((HID, 1), lambda b: (0, 0)),
            pl.BlockSpec((HID, HID), lambda b: (0, 0)),
            pl.BlockSpec((HID, 1), lambda b: (0, 0)),
            pl.BlockSpec((IN * HID, HID), lambda b: (0, 0)),
            pl.BlockSpec((IN * HID, 1), lambda b: (0, 0)),
        ],
        out_specs=pl.BlockSpec((B, HID), lambda b: (b, 0)),
        out_shape=jax.ShapeDtypeStruct((E_PAD, HID), _f32),
    )(eaT, xg, stats, W1.T, b1.reshape(HID, 1), W2.T, b2.reshape(HID, 1),
      W3.T, b3.reshape(IN * HID, 1))

    partials = _sc_scatter(msg, dst3, zeros_n)

    out = pl.pallas_call(
        _final_body,
        out_shape=jax.ShapeDtypeStruct((N, OUT), _f32),
    )(partials, r, L1, bL1.reshape(1, HID), L2, bL2.reshape(1, OUT))
    return out


# speedup vs baseline: 3.8781x; 1.0869x over previous
import functools

import jax
import jax.numpy as jnp
from jax import lax
from jax.experimental import pallas as pl
from jax.experimental.pallas import tpu as pltpu
from jax.experimental.pallas import tpu_sc as plsc

N = 10000
E = 160000
IN = 32
OUT = 32
HID = 32
ED = 16
EPS = 1e-5

NW = 32
CH = 128
E_PAD = 163840
CPW = E_PAD // (NW * CH)
B = 2048
G = 4
SB = B // G
GRID = E_PAD // B
P_ROWS = E_PAD // G

_f32 = jnp.float32


def _scan_order():
    p = jnp.arange(E_PAD, dtype=jnp.int32)
    pr, g = p // G, p % G
    return (pr // SB) * B + g * SB + (pr % SB)


def _stats_root_body(x_ref, root_ref, ncb_ref, stats_ref, r_ref):
    x = x_ref[...]
    mean = jnp.mean(x, axis=0, keepdims=True)
    xc = x - mean
    var = jnp.mean(xc * xc, axis=0, keepdims=True)
    istd = lax.rsqrt(var + EPS)
    xn = xc * istd
    r_ref[...] = jnp.dot(xn, root_ref[...], preferred_element_type=_f32) + ncb_ref[...]
    stats_ref[...] = jnp.concatenate(
        [mean, istd, jnp.zeros((6, IN), _f32)], axis=0)


def _msg_body(ea_ref, xgp_ref, stats_ref, w1_ref, b1_ref, w2_ref, b2_ref,
              w3t_ref, b3_ref, out_ref):
    b = pl.program_id(0)
    ea = ea_ref[...]
    h1 = jnp.maximum(jnp.dot(ea, w1_ref[...], preferred_element_type=_f32)
                     + b1_ref[...], 0.0)
    h2 = jnp.maximum(jnp.dot(h1, w2_ref[...], preferred_element_type=_f32)
                     + b2_ref[...], 0.0)
    h2T = h2.T

    stats = stats_ref[...]
    meanr = jnp.concatenate([stats[0:1, :]] * G, axis=1)
    istdr = jnp.concatenate([stats[1:2, :]] * G, axis=1)
    xgn = (xgp_ref[...] - meanr) * istdr
    xgn3 = xgn.T.reshape(G, IN, SB)

    accs = []
    for g in range(G):
        wTg = jnp.maximum(
            jnp.dot(w3t_ref[...], h2T[:, g * SB:(g + 1) * SB],
                    preferred_element_type=_f32) + b3_ref[...], 0.0)
        acc = jnp.zeros((HID, SB), _f32)
        for i in range(IN):
            row = jnp.broadcast_to(xgn3[g, i:i + 1, :], (HID, SB))
            acc = acc + wTg[HID * i:HID * (i + 1), :] * row
        accs.append(acc)
    msgT = jnp.concatenate(accs, axis=0)

    rid = lax.broadcasted_iota(jnp.int32, (G * HID, SB), 0) // HID
    cid = lax.broadcasted_iota(jnp.int32, (G * HID, SB), 1)
    eid = b * B + rid * SB + cid
    msgT = jnp.where(eid < E, msgT, 0.0)
    out_ref[...] = msgT.T


def _final_body(p_ref, r_ref, l1_ref, bl1_ref, l2_ref, bl2_ref, out_ref):
    agg = p_ref[0] + p_ref[1]
    o = jnp.maximum(agg + r_ref[...], 0.0)
    h = jnp.maximum(jnp.dot(o, l1_ref[...], preferred_element_type=_f32)
                    + bl1_ref[...], 0.0)
    out_ref[...] = jnp.dot(h, l2_ref[...], preferred_element_type=_f32) + bl2_ref[...]


def _sc_gather(x, src3):
    mesh = plsc.VectorSubcoreMesh(core_axis_name="c", subcore_axis_name="s")

    @functools.partial(
        pl.kernel, mesh=mesh,
        out_type=jax.ShapeDtypeStruct((E_PAD, IN), _f32),
        scratch_types=[
            pltpu.VMEM((CPW, CH), jnp.int32),
            pltpu.VMEM((CH, IN), _f32),
            pltpu.SemaphoreType.DMA,
        ],
        compiler_params=pltpu.CompilerParams(use_tc_tiling_on_sc=False),
    )
    def k(x_hbm, src_hbm, out_hbm, idx_v, rows_v, sem):
        wid = lax.axis_index("s") * 2 + lax.axis_index("c")
        pltpu.sync_copy(src_hbm.at[wid], idx_v)

        def chunk(c, _):
            pltpu.async_copy(x_hbm.at[idx_v.at[c]], rows_v, sem).wait()
            pltpu.sync_copy(rows_v, out_hbm.at[pl.ds((wid * CPW + c) * CH, CH)])
            return 0

        lax.fori_loop(0, CPW, chunk, 0)

    return k(x, src3)


def _sc_scatter(msg, dst3, zeros_n):
    mesh = plsc.VectorSubcoreMesh(core_axis_name="c", subcore_axis_name="s")
    stripe = N // 16

    @functools.partial(
        pl.kernel, mesh=mesh,
        out_type=jax.ShapeDtypeStruct((2, N, HID), _f32),
        scratch_types=[
            pltpu.VMEM_SHARED((N, HID), _f32),
            pltpu.VMEM((CPW, CH), jnp.int32),
            pltpu.VMEM((CH, HID), _f32),
        ],
        compiler_params=pltpu.CompilerParams(use_tc_tiling_on_sc=False),
    )
    def k(msg_hbm, dst_hbm, zeros_hbm, out_hbm, acc_sh, idx_v, m_v):
        cid = lax.axis_index("c")
        sid = lax.axis_index("s")
        wid = sid * 2 + cid
        pltpu.sync_copy(zeros_hbm.at[pl.ds(sid * stripe, stripe)],
                        acc_sh.at[pl.ds(sid * stripe, stripe)])
        pltpu.sync_copy(dst_hbm.at[wid], idx_v)
        plsc.subcore_barrier()

        def chunk(c, _):
            pltpu.sync_copy(msg_hbm.at[pl.ds((wid * CPW + c) * CH, CH)], m_v)
            pltpu.sync_copy(m_v, acc_sh.at[idx_v.at[c]], add=True)
            return 0

        lax.fori_loop(0, CPW, chunk, 0)
        plsc.subcore_barrier()
        pltpu.sync_copy(acc_sh.at[pl.ds(sid * stripe, stripe)],
                        out_hbm.at[cid, pl.ds(sid * stripe, stripe)])

    return k(msg, dst3, zeros_n)


def kernel(x, edge_index, edge_attr, W1, b1, W2, b2, W3, b3, root, ncb,
           L1, bL1, L2, bL2):
    pad = E_PAD - E
    order = _scan_order()
    src3 = jnp.pad(edge_index[0], (0, pad))[order].reshape(NW, CPW, CH)
    dst3 = jnp.pad(edge_index[1], (0, pad))[order].reshape(NW, CPW, CH)
    ea = jnp.pad(edge_attr, ((0, pad), (0, 0)))
    zeros_n = jnp.zeros((N, HID), _f32)

    stats, r = pl.pallas_call(
        _stats_root_body,
        out_shape=[jax.ShapeDtypeStruct((8, IN), _f32),
                   jax.ShapeDtypeStruct((N, HID), _f32)],
    )(x, root, ncb.reshape(1, HID))

    xgp = _sc_gather(x, src3).reshape(P_ROWS, G * IN)

    msgp = pl.pallas_call(
        _msg_body,
        grid=(GRID,),
        in_specs=[
            pl.BlockSpec((B, ED), lambda b: (b, 0)),
            pl.BlockSpec((SB, G * IN), lambda b: (b, 0)),
            pl.BlockSpec((8, IN), lambda b: (0, 0)),
            pl.BlockSpec((ED, HID), lambda b: (0, 0)),
            pl.BlockSpec((1, HID), lambda b: (0, 0)),
            pl.BlockSpec((HID, HID), lambda b: (0, 0)),
            pl.BlockSpec((1, HID), lambda b: (0, 0)),
            pl.BlockSpec((IN * HID, HID), lambda b: (0, 0)),
            pl.BlockSpec((IN * HID, 1), lambda b: (0, 0)),
        ],
        out_specs=pl.BlockSpec((SB, G * HID), lambda b: (b, 0)),
        out_shape=jax.ShapeDtypeStruct((P_ROWS, G * HID), _f32),
    )(ea, xgp, stats, W1, b1.reshape(1, HID), W2, b2.reshape(1, HID),
      W3.T, b3.reshape(IN * HID, 1))

    partials = _sc_scatter(msgp.reshape(E_PAD, HID), dst3, zeros_n)

    out = pl.pallas_call(
        _final_body,
        out_shape=jax.ShapeDtypeStruct((N, OUT), _f32),
    )(partials, r, L1, bL1.reshape(1, HID), L2, bL2.reshape(1, OUT))
    return out


# speedup vs baseline: 4.4420x; 1.1454x over previous
import functools

import jax
import jax.numpy as jnp
from jax import lax
from jax.experimental import pallas as pl
from jax.experimental.pallas import tpu as pltpu
from jax.experimental.pallas import tpu_sc as plsc

N = 10000
E = 160000
IN = 32
OUT = 32
HID = 32
ED = 16
EPS = 1e-5

NW = 32
CH = 128
E_PAD = 163840
CPW = E_PAD // (NW * CH)
B = 2048
G = 4
SB = B // G
GRID = (E + B - 1) // B
P_ROWS = E_PAD // G
N_ACC = N + 8

NH = 1
HB = E_PAD // B // NH
HE = E_PAD // NH
CPW_H = CPW // NH

_f32 = jnp.float32


def _to_scan_order(v):
    return v.reshape(E_PAD // B, G, SB).transpose(0, 2, 1).reshape(-1)


def _stats_root_body(x_ref, root_ref, ncb_ref, stats_ref, r_ref):
    x = x_ref[...]
    mean = jnp.mean(x, axis=0, keepdims=True)
    xc = x - mean
    var = jnp.mean(xc * xc, axis=0, keepdims=True)
    istd = lax.rsqrt(var + EPS)
    xn = xc * istd
    r_ref[...] = jnp.dot(xn, root_ref[...], preferred_element_type=_f32) + ncb_ref[...]
    stats_ref[...] = jnp.concatenate(
        [mean, istd, jnp.zeros((6, IN), _f32)], axis=0)


def _make_msg_body(off):
    def body(ea_ref, xgp_ref, stats_ref, w1_ref, b1_ref, w2_ref, b2_ref,
             w3t_ref, b3_ref, out_ref):
        b = pl.program_id(0) + off
        ea = ea_ref[...]
        h1 = jnp.maximum(jnp.dot(ea, w1_ref[...], preferred_element_type=_f32)
                         + b1_ref[...], 0.0)
        h2 = jnp.maximum(jnp.dot(h1, w2_ref[...], preferred_element_type=_f32)
                         + b2_ref[...], 0.0)
        h2T = h2.T

        stats = stats_ref[...]
        meanr = jnp.concatenate([stats[0:1, :]] * G, axis=1)
        istdr = jnp.concatenate([stats[1:2, :]] * G, axis=1)
        xgn = (xgp_ref[...] - meanr) * istdr
        xgn3 = xgn.T.reshape(G, IN, SB)

        accs = []
        for g in range(G):
            wTg = jnp.maximum(
                jnp.dot(w3t_ref[...], h2T[:, g * SB:(g + 1) * SB],
                        preferred_element_type=_f32) + b3_ref[...], 0.0)
            acc = jnp.zeros((HID, SB), _f32)
            for i in range(IN):
                row = jnp.broadcast_to(xgn3[g, i:i + 1, :], (HID, SB))
                acc = acc + wTg[HID * i:HID * (i + 1), :] * row
            accs.append(acc)
        msgT = jnp.concatenate(accs, axis=0)

        rid = lax.broadcasted_iota(jnp.int32, (G * HID, SB), 0) // HID
        cid = lax.broadcasted_iota(jnp.int32, (G * HID, SB), 1)
        eid = b * B + rid * SB + cid
        msgT = jnp.where(eid < E, msgT, 0.0)
        out_ref[...] = msgT.T
    return body


def _final_body(p0_ref, r_ref, l1_ref, bl1_ref, l2_ref, bl2_ref, out_ref):
    agg = p0_ref[0] + p0_ref[1]
    o = jnp.maximum(agg + r_ref[...], 0.0)
    h = jnp.maximum(jnp.dot(o, l1_ref[...], preferred_element_type=_f32)
                    + bl1_ref[...], 0.0)
    out_ref[...] = jnp.dot(h, l2_ref[...], preferred_element_type=_f32) + bl2_ref[...]


def _sc_gather(x, src3):
    mesh = plsc.VectorSubcoreMesh(core_axis_name="c", subcore_axis_name="s")

    nb = 4
    nbat = CPW_H // nb

    @functools.partial(
        pl.kernel, mesh=mesh,
        out_type=jax.ShapeDtypeStruct((HE, IN), _f32),
        scratch_types=[
            pltpu.VMEM((CPW_H, CH), jnp.int32),
            pltpu.VMEM((2, nb * CH, IN), _f32),
            pltpu.SemaphoreType.DMA,
            pltpu.SemaphoreType.DMA,
            pltpu.SemaphoreType.DMA,
        ],
        compiler_params=pltpu.CompilerParams(use_tc_tiling_on_sc=False),
    )
    def k(x_hbm, src_hbm, out_hbm, idx_v, rows_v, sem_g, sem_w0, sem_w1):
        wid = lax.axis_index("s") * 2 + lax.axis_index("c")
        pltpu.sync_copy(src_hbm.at[wid], idx_v)
        sem_w = (sem_w0, sem_w1)
        hw = [None, None]
        for bat in range(nbat):
            p = bat % 2
            if hw[p] is not None:
                hw[p].wait()
            hg = [
                pltpu.async_copy(x_hbm.at[idx_v.at[bat * nb + j]],
                                 rows_v.at[p, pl.ds(j * CH, CH)], sem_g)
                for j in range(nb)
            ]
            for h in hg:
                h.wait()
            hw[p] = pltpu.async_copy(
                rows_v.at[p],
                out_hbm.at[pl.ds(wid * CPW_H * CH + bat * nb * CH, nb * CH)],
                sem_w[p])
        hw[0].wait()
        hw[1].wait()

    return k(x, src3)


def _sc_scatter(msg, dst3, zeros_n):
    mesh = plsc.VectorSubcoreMesh(core_axis_name="c", subcore_axis_name="s")
    stripe = N // 16

    nb = 4
    nbat = CPW_H // nb

    @functools.partial(
        pl.kernel, mesh=mesh,
        out_type=jax.ShapeDtypeStruct((2, N, HID), _f32),
        scratch_types=[
            pltpu.VMEM_SHARED((N_ACC, HID), _f32),
            pltpu.VMEM((CPW_H, CH), jnp.int32),
            pltpu.VMEM((2, nb * CH, HID), _f32),
            pltpu.SemaphoreType.DMA,
            pltpu.SemaphoreType.DMA,
        ],
        compiler_params=pltpu.CompilerParams(use_tc_tiling_on_sc=False),
    )
    def k(msg_hbm, dst_hbm, zeros_hbm, out_hbm, acc_sh, idx_v, m_v, sem0, sem1):
        cid = lax.axis_index("c")
        sid = lax.axis_index("s")
        wid = sid * 2 + cid
        base = wid * CPW_H * CH
        pltpu.sync_copy(zeros_hbm.at[pl.ds(sid * stripe, stripe)],
                        acc_sh.at[pl.ds(sid * stripe, stripe)])
        pltpu.sync_copy(dst_hbm.at[wid], idx_v)
        plsc.subcore_barrier()
        sems = (sem0, sem1)
        hl = pltpu.async_copy(msg_hbm.at[pl.ds(base, nb * CH)],
                              m_v.at[0], sems[0])
        for bat in range(nbat):
            p = bat % 2
            hl.wait()
            if bat + 1 < nbat:
                hl = pltpu.async_copy(
                    msg_hbm.at[pl.ds(base + (bat + 1) * nb * CH, nb * CH)],
                    m_v.at[1 - p], sems[1 - p])
            for j in range(nb):
                pltpu.sync_copy(m_v.at[p, pl.ds(j * CH, CH)],
                                acc_sh.at[idx_v.at[bat * nb + j]], add=True)
        plsc.subcore_barrier()
        pltpu.sync_copy(acc_sh.at[pl.ds(sid * stripe, stripe)],
                        out_hbm.at[cid, pl.ds(sid * stripe, stripe)])

    return k(msg, dst3, zeros_n)


def _msg_call(half, ea, xgp, stats, W1, b1, W2, b2, W3T, b3):
    off = half * HB
    grid_n = min(GRID - off, HB)
    return pl.pallas_call(
        _make_msg_body(off),
        grid=(grid_n,),
        in_specs=[
            pl.BlockSpec((B, ED), lambda b: (b + off, 0)),
            pl.BlockSpec((SB, G * IN), lambda b: (b, 0)),
            pl.BlockSpec((8, IN), lambda b: (0, 0)),
            pl.BlockSpec((ED, HID), lambda b: (0, 0)),
            pl.BlockSpec((1, HID), lambda b: (0, 0)),
            pl.BlockSpec((HID, HID), lambda b: (0, 0)),
            pl.BlockSpec((1, HID), lambda b: (0, 0)),
            pl.BlockSpec((IN * HID, HID), lambda b: (0, 0)),
            pl.BlockSpec((IN * HID, 1), lambda b: (0, 0)),
        ],
        out_specs=pl.BlockSpec((SB, G * HID), lambda b: (b, 0)),
        out_shape=jax.ShapeDtypeStruct((HE // G, G * HID), _f32),
    )(ea, xgp, stats, W1, b1, W2, b2, W3T, b3)


def kernel(x, edge_index, edge_attr, W1, b1, W2, b2, W3, b3, root, ncb,
           L1, bL1, L2, bL2):
    pad = E_PAD - E
    src_scan = _to_scan_order(jnp.pad(edge_index[0], (0, pad)))
    dst_scan = _to_scan_order(jnp.pad(edge_index[1], (0, pad),
                                      constant_values=jnp.int32(N)))
    src3 = src_scan.reshape(NH, NW, CPW_H, CH)
    dst3 = dst_scan.reshape(NH, NW, CPW_H, CH)
    zeros_n = jnp.zeros((N, HID), _f32)
    b1r, b2r = b1.reshape(1, HID), b2.reshape(1, HID)
    W3T = W3.T
    b3c = b3.reshape(IN * HID, 1)

    stats, r = pl.pallas_call(
        _stats_root_body,
        out_shape=[jax.ShapeDtypeStruct((8, IN), _f32),
                   jax.ShapeDtypeStruct((N, HID), _f32)],
    )(x, root, ncb.reshape(1, HID))

    xg0 = _sc_gather(x, src3[0]).reshape(HE // G, G * IN)
    msg0 = _msg_call(0, edge_attr, xg0, stats, W1, b1r, W2, b2r, W3T, b3c)
    p0 = _sc_scatter(msg0.reshape(HE, HID), dst3[0], zeros_n)

    out = pl.pallas_call(
        _final_body,
        out_shape=jax.ShapeDtypeStruct((N, OUT), _f32),
    )(p0, r, L1, bL1.reshape(1, HID), L2, bL2.reshape(1, OUT))
    return out


# speedup vs baseline: 4.4548x; 1.0029x over previous
import functools

import jax
import jax.numpy as jnp
from jax import lax
from jax.experimental import pallas as pl
from jax.experimental.pallas import tpu as pltpu
from jax.experimental.pallas import tpu_sc as plsc

N = 10000
E = 160000
IN = 32
OUT = 32
HID = 32
ED = 16
EPS = 1e-5

NW = 32
CH = 128
E_PAD = 163840
CPW = E_PAD // (NW * CH)
B = 2048
G = 4
SB = B // G
GRID = (E + B - 1) // B
P_ROWS = E_PAD // G
N_ACC = N + 8

NH = 1
HB = E_PAD // B // NH
HE = E_PAD // NH
CPW_H = CPW // NH

_f32 = jnp.float32


def _to_scan_order(v):
    return v.reshape(E_PAD // B, G, SB).transpose(0, 2, 1).reshape(-1)


def _stats_root_body(x_ref, root_ref, ncb_ref, stats_ref, r_ref):
    x = x_ref[...]
    mean = jnp.mean(x, axis=0, keepdims=True)
    xc = x - mean
    var = jnp.mean(xc * xc, axis=0, keepdims=True)
    istd = lax.rsqrt(var + EPS)
    xn = xc * istd
    r_ref[...] = jnp.dot(xn, root_ref[...], preferred_element_type=_f32) + ncb_ref[...]
    stats_ref[...] = jnp.concatenate(
        [mean, istd, jnp.zeros((6, IN), _f32)], axis=0)


def _make_msg_body(off):
    def body(ea_ref, xgp_ref, stats_ref, w1_ref, b1_ref, w2_ref, b2_ref,
             w3t_ref, b3_ref, out_ref):
        b = pl.program_id(0) + off
        ea = ea_ref[...]
        h1 = jnp.maximum(jnp.dot(ea, w1_ref[...], preferred_element_type=_f32)
                         + b1_ref[...], 0.0)
        h2 = jnp.maximum(jnp.dot(h1, w2_ref[...], preferred_element_type=_f32)
                         + b2_ref[...], 0.0)
        h2T = h2.T

        stats = stats_ref[...]
        meanr = jnp.concatenate([stats[0:1, :]] * G, axis=1)
        istdr = jnp.concatenate([stats[1:2, :]] * G, axis=1)
        xgn = (xgp_ref[...] - meanr) * istdr
        xgn3 = xgn.T.reshape(G, IN, SB)

        accs = []
        for g in range(G):
            wTg = jnp.maximum(
                jnp.dot(w3t_ref[...], h2T[:, g * SB:(g + 1) * SB],
                        preferred_element_type=_f32) + b3_ref[...], 0.0)
            acc = jnp.zeros((HID, SB), _f32)
            for i in range(IN):
                row = jnp.broadcast_to(xgn3[g, i:i + 1, :], (HID, SB))
                acc = acc + wTg[HID * i:HID * (i + 1), :] * row
            accs.append(acc)
        msgT = jnp.concatenate(accs, axis=0)

        rid = lax.broadcasted_iota(jnp.int32, (G * HID, SB), 0) // HID
        cid = lax.broadcasted_iota(jnp.int32, (G * HID, SB), 1)
        eid = b * B + rid * SB + cid
        msgT = jnp.where(eid < E, msgT, 0.0)
        out_ref[...] = msgT.T
    return body


def _final_body(p0_ref, r_ref, l1_ref, bl1_ref, l2_ref, bl2_ref, out_ref):
    agg = p0_ref[0] + p0_ref[1]
    o = jnp.maximum(agg + r_ref[...], 0.0)
    h = jnp.maximum(jnp.dot(o, l1_ref[...], preferred_element_type=_f32)
                    + bl1_ref[...], 0.0)
    out_ref[...] = jnp.dot(h, l2_ref[...], preferred_element_type=_f32) + bl2_ref[...]


def _sc_gather(x, src3):
    mesh = plsc.VectorSubcoreMesh(core_axis_name="c", subcore_axis_name="s")

    nb = 8
    nbat = CPW_H // nb

    @functools.partial(
        pl.kernel, mesh=mesh,
        out_type=jax.ShapeDtypeStruct((HE, IN), _f32),
        scratch_types=[
            pltpu.VMEM((CPW_H, CH), jnp.int32),
            pltpu.VMEM((2, nb * CH, IN), _f32),
            pltpu.SemaphoreType.DMA,
            pltpu.SemaphoreType.DMA,
            pltpu.SemaphoreType.DMA,
        ],
        compiler_params=pltpu.CompilerParams(use_tc_tiling_on_sc=False),
    )
    def k(x_hbm, src_hbm, out_hbm, idx_v, rows_v, sem_g, sem_w0, sem_w1):
        wid = lax.axis_index("s") * 2 + lax.axis_index("c")
        pltpu.sync_copy(src_hbm.at[wid], idx_v)
        sem_w = (sem_w0, sem_w1)
        hw = [None, None]
        for bat in range(nbat):
            p = bat % 2
            if hw[p] is not None:
                hw[p].wait()
            hg = [
                pltpu.async_copy(x_hbm.at[idx_v.at[bat * nb + j]],
                                 rows_v.at[p, pl.ds(j * CH, CH)], sem_g)
                for j in range(nb)
            ]
            for h in hg:
                h.wait()
            hw[p] = pltpu.async_copy(
                rows_v.at[p],
                out_hbm.at[pl.ds(wid * CPW_H * CH + bat * nb * CH, nb * CH)],
                sem_w[p])
        hw[0].wait()
        hw[1].wait()

    return k(x, src3)


def _sc_scatter(msg, dst3, zeros_n):
    mesh = plsc.VectorSubcoreMesh(core_axis_name="c", subcore_axis_name="s")
    stripe = N // 16

    nb = 8
    nbat = CPW_H // nb

    @functools.partial(
        pl.kernel, mesh=mesh,
        out_type=jax.ShapeDtypeStruct((2, N, HID), _f32),
        scratch_types=[
            pltpu.VMEM_SHARED((N_ACC, HID), _f32),
            pltpu.VMEM((CPW_H, CH), jnp.int32),
            pltpu.VMEM((2, nb * CH, HID), _f32),
            pltpu.SemaphoreType.DMA,
            pltpu.SemaphoreType.DMA,
        ],
        compiler_params=pltpu.CompilerParams(use_tc_tiling_on_sc=False),
    )
    def k(msg_hbm, dst_hbm, zeros_hbm, out_hbm, acc_sh, idx_v, m_v, sem0, sem1):
        cid = lax.axis_index("c")
        sid = lax.axis_index("s")
        wid = sid * 2 + cid
        base = wid * CPW_H * CH
        pltpu.sync_copy(zeros_hbm.at[pl.ds(sid * stripe, stripe)],
                        acc_sh.at[pl.ds(sid * stripe, stripe)])
        pltpu.sync_copy(dst_hbm.at[wid], idx_v)
        plsc.subcore_barrier()
        sems = (sem0, sem1)
        hl = pltpu.async_copy(msg_hbm.at[pl.ds(base, nb * CH)],
                              m_v.at[0], sems[0])
        for bat in range(nbat):
            p = bat % 2
            hl.wait()
            if bat + 1 < nbat:
                hl = pltpu.async_copy(
                    msg_hbm.at[pl.ds(base + (bat + 1) * nb * CH, nb * CH)],
                    m_v.at[1 - p], sems[1 - p])
            for j in range(nb):
                pltpu.sync_copy(m_v.at[p, pl.ds(j * CH, CH)],
                                acc_sh.at[idx_v.at[bat * nb + j]], add=True)
        plsc.subcore_barrier()
        pltpu.sync_copy(acc_sh.at[pl.ds(sid * stripe, stripe)],
                        out_hbm.at[cid, pl.ds(sid * stripe, stripe)])

    return k(msg, dst3, zeros_n)


def _msg_call(half, ea, xgp, stats, W1, b1, W2, b2, W3T, b3):
    off = half * HB
    grid_n = min(GRID - off, HB)
    return pl.pallas_call(
        _make_msg_body(off),
        grid=(grid_n,),
        in_specs=[
            pl.BlockSpec((B, ED), lambda b: (b + off, 0)),
            pl.BlockSpec((SB, G * IN), lambda b: (b, 0)),
            pl.BlockSpec((8, IN), lambda b: (0, 0)),
            pl.BlockSpec((ED, HID), lambda b: (0, 0)),
            pl.BlockSpec((1, HID), lambda b: (0, 0)),
            pl.BlockSpec((HID, HID), lambda b: (0, 0)),
            pl.BlockSpec((1, HID), lambda b: (0, 0)),
            pl.BlockSpec((IN * HID, HID), lambda b: (0, 0)),
            pl.BlockSpec((IN * HID, 1), lambda b: (0, 0)),
        ],
        out_specs=pl.BlockSpec((SB, G * HID), lambda b: (b, 0)),
        out_shape=jax.ShapeDtypeStruct((HE // G, G * HID), _f32),
    )(ea, xgp, stats, W1, b1, W2, b2, W3T, b3)


def kernel(x, edge_index, edge_attr, W1, b1, W2, b2, W3, b3, root, ncb,
           L1, bL1, L2, bL2):
    pad = E_PAD - E
    src_scan = _to_scan_order(jnp.pad(edge_index[0], (0, pad)))
    dst_scan = _to_scan_order(jnp.pad(edge_index[1], (0, pad),
                                      constant_values=jnp.int32(N)))
    src3 = src_scan.reshape(NH, NW, CPW_H, CH)
    dst3 = dst_scan.reshape(NH, NW, CPW_H, CH)
    zeros_n = jnp.zeros((N, HID), _f32)
    b1r, b2r = b1.reshape(1, HID), b2.reshape(1, HID)
    W3T = W3.T
    b3c = b3.reshape(IN * HID, 1)

    stats, r = pl.pallas_call(
        _stats_root_body,
        out_shape=[jax.ShapeDtypeStruct((8, IN), _f32),
                   jax.ShapeDtypeStruct((N, HID), _f32)],
    )(x, root, ncb.reshape(1, HID))

    xg0 = _sc_gather(x, src3[0]).reshape(HE // G, G * IN)
    msg0 = _msg_call(0, edge_attr, xg0, stats, W1, b1r, W2, b2r, W3T, b3c)
    p0 = _sc_scatter(msg0.reshape(HE, HID), dst3[0], zeros_n)

    out = pl.pallas_call(
        _final_body,
        out_shape=jax.ShapeDtypeStruct((N, OUT), _f32),
    )(p0, r, L1, bL1.reshape(1, HID), L2, bL2.reshape(1, OUT))
    return out


# speedup vs baseline: 4.5591x; 1.0234x over previous
import functools

import jax
import jax.numpy as jnp
from jax import lax
from jax.experimental import pallas as pl
from jax.experimental.pallas import tpu as pltpu
from jax.experimental.pallas import tpu_sc as plsc

N = 10000
E = 160000
IN = 32
OUT = 32
HID = 32
ED = 16
EPS = 1e-5

NW = 32
CH = 128
E_PAD = 163840
CPW = E_PAD // (NW * CH)
B = 4096
G = 4
SB = B // G
GRID = (E + B - 1) // B
P_ROWS = E_PAD // G
N_ACC = N + 8

NH = 1
HB = E_PAD // B // NH
HE = E_PAD // NH
CPW_H = CPW // NH

_f32 = jnp.float32


def _to_scan_order(v):
    return v.reshape(E_PAD // B, G, SB).transpose(0, 2, 1).reshape(-1)


def _stats_root_body(x_ref, root_ref, ncb_ref, stats_ref, r_ref):
    x = x_ref[...]
    mean = jnp.mean(x, axis=0, keepdims=True)
    xc = x - mean
    var = jnp.mean(xc * xc, axis=0, keepdims=True)
    istd = lax.rsqrt(var + EPS)
    xn = xc * istd
    r_ref[...] = jnp.dot(xn, root_ref[...], preferred_element_type=_f32) + ncb_ref[...]
    stats_ref[...] = jnp.concatenate(
        [mean, istd, jnp.zeros((6, IN), _f32)], axis=0)


def _make_msg_body(off):
    def body(ea_ref, xgp_ref, stats_ref, w1_ref, b1_ref, w2_ref, b2_ref,
             w3t_ref, b3_ref, out_ref):
        b = pl.program_id(0) + off
        ea = ea_ref[...]
        h1 = jnp.maximum(jnp.dot(ea, w1_ref[...], preferred_element_type=_f32)
                         + b1_ref[...], 0.0)
        h2 = jnp.maximum(jnp.dot(h1, w2_ref[...], preferred_element_type=_f32)
                         + b2_ref[...], 0.0)
        h2T = h2.T

        stats = stats_ref[...]
        meanr = jnp.concatenate([stats[0:1, :]] * G, axis=1)
        istdr = jnp.concatenate([stats[1:2, :]] * G, axis=1)
        xgn = (xgp_ref[...] - meanr) * istdr
        xgn3 = xgn.T.reshape(G, IN, SB)

        accs = []
        for g in range(G):
            wTg = jnp.maximum(
                jnp.dot(w3t_ref[...], h2T[:, g * SB:(g + 1) * SB],
                        preferred_element_type=_f32) + b3_ref[...], 0.0)
            acc = jnp.zeros((HID, SB), _f32)
            for i in range(IN):
                row = jnp.broadcast_to(xgn3[g, i:i + 1, :], (HID, SB))
                acc = acc + wTg[HID * i:HID * (i + 1), :] * row
            accs.append(acc)
        msgT = jnp.concatenate(accs, axis=0)

        rid = lax.broadcasted_iota(jnp.int32, (G * HID, SB), 0) // HID
        cid = lax.broadcasted_iota(jnp.int32, (G * HID, SB), 1)
        eid = b * B + rid * SB + cid
        msgT = jnp.where(eid < E, msgT, 0.0)
        out_ref[...] = msgT.T
    return body


def _final_body(p0_ref, r_ref, l1_ref, bl1_ref, l2_ref, bl2_ref, out_ref):
    agg = p0_ref[0] + p0_ref[1]
    o = jnp.maximum(agg + r_ref[...], 0.0)
    h = jnp.maximum(jnp.dot(o, l1_ref[...], preferred_element_type=_f32)
                    + bl1_ref[...], 0.0)
    out_ref[...] = jnp.dot(h, l2_ref[...], preferred_element_type=_f32) + bl2_ref[...]


def _sc_gather(x, src3):
    mesh = plsc.VectorSubcoreMesh(core_axis_name="c", subcore_axis_name="s")

    nb = 8
    nbat = CPW_H // nb

    @functools.partial(
        pl.kernel, mesh=mesh,
        out_type=jax.ShapeDtypeStruct((HE, IN), _f32),
        scratch_types=[
            pltpu.VMEM((CPW_H, CH), jnp.int32),
            pltpu.VMEM((2, nb * CH, IN), _f32),
            pltpu.SemaphoreType.DMA,
            pltpu.SemaphoreType.DMA,
            pltpu.SemaphoreType.DMA,
        ],
        compiler_params=pltpu.CompilerParams(use_tc_tiling_on_sc=False),
    )
    def k(x_hbm, src_hbm, out_hbm, idx_v, rows_v, sem_g, sem_w0, sem_w1):
        wid = lax.axis_index("s") * 2 + lax.axis_index("c")
        pltpu.sync_copy(src_hbm.at[wid], idx_v)
        sem_w = (sem_w0, sem_w1)
        hw = [None, None]
        for bat in range(nbat):
            p = bat % 2
            if hw[p] is not None:
                hw[p].wait()
            hg = [
                pltpu.async_copy(x_hbm.at[idx_v.at[bat * nb + j]],
                                 rows_v.at[p, pl.ds(j * CH, CH)], sem_g)
                for j in range(nb)
            ]
            for h in hg:
                h.wait()
            hw[p] = pltpu.async_copy(
                rows_v.at[p],
                out_hbm.at[pl.ds(wid * CPW_H * CH + bat * nb * CH, nb * CH)],
                sem_w[p])
        hw[0].wait()
        hw[1].wait()

    return k(x, src3)


def _sc_scatter(msg, dst3, zeros_n):
    mesh = plsc.VectorSubcoreMesh(core_axis_name="c", subcore_axis_name="s")
    stripe = N // 16

    nb = 8
    nbat = CPW_H // nb

    @functools.partial(
        pl.kernel, mesh=mesh,
        out_type=jax.ShapeDtypeStruct((2, N, HID), _f32),
        scratch_types=[
            pltpu.VMEM_SHARED((N_ACC, HID), _f32),
            pltpu.VMEM((CPW_H, CH), jnp.int32),
            pltpu.VMEM((2, nb * CH, HID), _f32),
            pltpu.SemaphoreType.DMA,
            pltpu.SemaphoreType.DMA,
        ],
        compiler_params=pltpu.CompilerParams(use_tc_tiling_on_sc=False),
    )
    def k(msg_hbm, dst_hbm, zeros_hbm, out_hbm, acc_sh, idx_v, m_v, sem0, sem1):
        cid = lax.axis_index("c")
        sid = lax.axis_index("s")
        wid = sid * 2 + cid
        base = wid * CPW_H * CH
        pltpu.sync_copy(zeros_hbm.at[pl.ds(sid * stripe, stripe)],
                        acc_sh.at[pl.ds(sid * stripe, stripe)])
        pltpu.sync_copy(dst_hbm.at[wid], idx_v)
        plsc.subcore_barrier()
        sems = (sem0, sem1)
        hl = pltpu.async_copy(msg_hbm.at[pl.ds(base, nb * CH)],
                              m_v.at[0], sems[0])
        for bat in range(nbat):
            p = bat % 2
            hl.wait()
            if bat + 1 < nbat:
                hl = pltpu.async_copy(
                    msg_hbm.at[pl.ds(base + (bat + 1) * nb * CH, nb * CH)],
                    m_v.at[1 - p], sems[1 - p])
            for j in range(nb):
                pltpu.sync_copy(m_v.at[p, pl.ds(j * CH, CH)],
                                acc_sh.at[idx_v.at[bat * nb + j]], add=True)
        plsc.subcore_barrier()
        pltpu.sync_copy(acc_sh.at[pl.ds(sid * stripe, stripe)],
                        out_hbm.at[cid, pl.ds(sid * stripe, stripe)])

    return k(msg, dst3, zeros_n)


def _msg_call(half, ea, xgp, stats, W1, b1, W2, b2, W3T, b3):
    off = half * HB
    grid_n = min(GRID - off, HB)
    return pl.pallas_call(
        _make_msg_body(off),
        grid=(grid_n,),
        in_specs=[
            pl.BlockSpec((B, ED), lambda b: (b + off, 0)),
            pl.BlockSpec((SB, G * IN), lambda b: (b, 0)),
            pl.BlockSpec((8, IN), lambda b: (0, 0)),
            pl.BlockSpec((ED, HID), lambda b: (0, 0)),
            pl.BlockSpec((1, HID), lambda b: (0, 0)),
            pl.BlockSpec((HID, HID), lambda b: (0, 0)),
            pl.BlockSpec((1, HID), lambda b: (0, 0)),
            pl.BlockSpec((IN * HID, HID), lambda b: (0, 0)),
            pl.BlockSpec((IN * HID, 1), lambda b: (0, 0)),
        ],
        out_specs=pl.BlockSpec((SB, G * HID), lambda b: (b, 0)),
        out_shape=jax.ShapeDtypeStruct((HE // G, G * HID), _f32),
    )(ea, xgp, stats, W1, b1, W2, b2, W3T, b3)


def kernel(x, edge_index, edge_attr, W1, b1, W2, b2, W3, b3, root, ncb,
           L1, bL1, L2, bL2):
    pad = E_PAD - E
    src_scan = _to_scan_order(jnp.pad(edge_index[0], (0, pad)))
    dst_scan = _to_scan_order(jnp.pad(edge_index[1], (0, pad),
                                      constant_values=jnp.int32(N)))
    src3 = src_scan.reshape(NH, NW, CPW_H, CH)
    dst3 = dst_scan.reshape(NH, NW, CPW_H, CH)
    zeros_n = jnp.zeros((N, HID), _f32)
    b1r, b2r = b1.reshape(1, HID), b2.reshape(1, HID)
    W3T = W3.T
    b3c = b3.reshape(IN * HID, 1)

    stats, r = pl.pallas_call(
        _stats_root_body,
        out_shape=[jax.ShapeDtypeStruct((8, IN), _f32),
                   jax.ShapeDtypeStruct((N, HID), _f32)],
    )(x, root, ncb.reshape(1, HID))

    xg0 = _sc_gather(x, src3[0]).reshape(HE // G, G * IN)
    msg0 = _msg_call(0, edge_attr, xg0, stats, W1, b1r, W2, b2r, W3T, b3c)
    p0 = _sc_scatter(msg0.reshape(HE, HID), dst3[0], zeros_n)

    out = pl.pallas_call(
        _final_body,
        out_shape=jax.ShapeDtypeStruct((N, OUT), _f32),
    )(p0, r, L1, bL1.reshape(1, HID), L2, bL2.reshape(1, OUT))
    return out


# speedup vs baseline: 4.6172x; 1.0127x over previous
import functools

import jax
import jax.numpy as jnp
from jax import lax
from jax.experimental import pallas as pl
from jax.experimental.pallas import tpu as pltpu
from jax.experimental.pallas import tpu_sc as plsc

N = 10000
E = 160000
IN = 32
OUT = 32
HID = 32
ED = 16
EPS = 1e-5

NW = 32
CH = 128
E_PAD = 163840
CPW = E_PAD // (NW * CH)
B = 8192
G = 4
SB = B // G
GRID = (E + B - 1) // B
P_ROWS = E_PAD // G
N_ACC = N + 8

NH = 1
HB = E_PAD // B // NH
HE = E_PAD // NH
CPW_H = CPW // NH

_f32 = jnp.float32


def _to_scan_order(v):
    return v.reshape(E_PAD // B, G, SB).transpose(0, 2, 1).reshape(-1)


def _stats_root_body(x_ref, root_ref, ncb_ref, stats_ref, r_ref):
    x = x_ref[...]
    mean = jnp.mean(x, axis=0, keepdims=True)
    xc = x - mean
    var = jnp.mean(xc * xc, axis=0, keepdims=True)
    istd = lax.rsqrt(var + EPS)
    xn = xc * istd
    r_ref[...] = jnp.dot(xn, root_ref[...], preferred_element_type=_f32) + ncb_ref[...]
    stats_ref[...] = jnp.concatenate(
        [mean, istd, jnp.zeros((6, IN), _f32)], axis=0)


def _make_msg_body(off):
    def body(ea_ref, xgp_ref, stats_ref, w1_ref, b1_ref, w2_ref, b2_ref,
             w3t_ref, b3_ref, out_ref):
        b = pl.program_id(0) + off
        ea = ea_ref[...]
        h1 = jnp.maximum(jnp.dot(ea, w1_ref[...], preferred_element_type=_f32)
                         + b1_ref[...], 0.0)
        h2 = jnp.maximum(jnp.dot(h1, w2_ref[...], preferred_element_type=_f32)
                         + b2_ref[...], 0.0)
        h2T = h2.T

        stats = stats_ref[...]
        meanr = jnp.concatenate([stats[0:1, :]] * G, axis=1)
        istdr = jnp.concatenate([stats[1:2, :]] * G, axis=1)
        xgn = (xgp_ref[...] - meanr) * istdr
        xgn3 = xgn.T.reshape(G, IN, SB)

        accs = []
        for g in range(G):
            wTg = jnp.maximum(
                jnp.dot(w3t_ref[...], h2T[:, g * SB:(g + 1) * SB],
                        preferred_element_type=_f32) + b3_ref[...], 0.0)
            acc = jnp.zeros((HID, SB), _f32)
            for i in range(IN):
                row = jnp.broadcast_to(xgn3[g, i:i + 1, :], (HID, SB))
                acc = acc + wTg[HID * i:HID * (i + 1), :] * row
            accs.append(acc)
        msgT = jnp.concatenate(accs, axis=0)

        rid = lax.broadcasted_iota(jnp.int32, (G * HID, SB), 0) // HID
        cid = lax.broadcasted_iota(jnp.int32, (G * HID, SB), 1)
        eid = b * B + rid * SB + cid
        msgT = jnp.where(eid < E, msgT, 0.0)
        out_ref[...] = msgT.T
    return body


def _final_body(p0_ref, r_ref, l1_ref, bl1_ref, l2_ref, bl2_ref, out_ref):
    agg = p0_ref[0] + p0_ref[1]
    o = jnp.maximum(agg + r_ref[...], 0.0)
    h = jnp.maximum(jnp.dot(o, l1_ref[...], preferred_element_type=_f32)
                    + bl1_ref[...], 0.0)
    out_ref[...] = jnp.dot(h, l2_ref[...], preferred_element_type=_f32) + bl2_ref[...]


def _sc_gather(x, src3):
    mesh = plsc.VectorSubcoreMesh(core_axis_name="c", subcore_axis_name="s")

    nb = 8
    nbat = CPW_H // nb

    @functools.partial(
        pl.kernel, mesh=mesh,
        out_type=jax.ShapeDtypeStruct((HE, IN), _f32),
        scratch_types=[
            pltpu.VMEM((CPW_H, CH), jnp.int32),
            pltpu.VMEM((2, nb * CH, IN), _f32),
            pltpu.SemaphoreType.DMA,
            pltpu.SemaphoreType.DMA,
            pltpu.SemaphoreType.DMA,
        ],
        compiler_params=pltpu.CompilerParams(use_tc_tiling_on_sc=False),
    )
    def k(x_hbm, src_hbm, out_hbm, idx_v, rows_v, sem_g, sem_w0, sem_w1):
        wid = lax.axis_index("s") * 2 + lax.axis_index("c")
        pltpu.sync_copy(src_hbm.at[wid], idx_v)
        sem_w = (sem_w0, sem_w1)
        hw = [None, None]
        for bat in range(nbat):
            p = bat % 2
            if hw[p] is not None:
                hw[p].wait()
            hg = [
                pltpu.async_copy(x_hbm.at[idx_v.at[bat * nb + j]],
                                 rows_v.at[p, pl.ds(j * CH, CH)], sem_g)
                for j in range(nb)
            ]
            for h in hg:
                h.wait()
            hw[p] = pltpu.async_copy(
                rows_v.at[p],
                out_hbm.at[pl.ds(wid * CPW_H * CH + bat * nb * CH, nb * CH)],
                sem_w[p])
        hw[0].wait()
        hw[1].wait()

    return k(x, src3)


def _sc_scatter(msg, dst3, zeros_n):
    mesh = plsc.VectorSubcoreMesh(core_axis_name="c", subcore_axis_name="s")
    stripe = N // 16

    nb = 8
    nbat = CPW_H // nb

    @functools.partial(
        pl.kernel, mesh=mesh,
        out_type=jax.ShapeDtypeStruct((2, N, HID), _f32),
        scratch_types=[
            pltpu.VMEM_SHARED((N_ACC, HID), _f32),
            pltpu.VMEM((CPW_H, CH), jnp.int32),
            pltpu.VMEM((2, nb * CH, HID), _f32),
            pltpu.SemaphoreType.DMA,
            pltpu.SemaphoreType.DMA,
        ],
        compiler_params=pltpu.CompilerParams(use_tc_tiling_on_sc=False),
    )
    def k(msg_hbm, dst_hbm, zeros_hbm, out_hbm, acc_sh, idx_v, m_v, sem0, sem1):
        cid = lax.axis_index("c")
        sid = lax.axis_index("s")
        wid = sid * 2 + cid
        base = wid * CPW_H * CH
        pltpu.sync_copy(zeros_hbm.at[pl.ds(sid * stripe, stripe)],
                        acc_sh.at[pl.ds(sid * stripe, stripe)])
        pltpu.sync_copy(dst_hbm.at[wid], idx_v)
        plsc.subcore_barrier()
        sems = (sem0, sem1)
        hl = pltpu.async_copy(msg_hbm.at[pl.ds(base, nb * CH)],
                              m_v.at[0], sems[0])
        for bat in range(nbat):
            p = bat % 2
            hl.wait()
            if bat + 1 < nbat:
                hl = pltpu.async_copy(
                    msg_hbm.at[pl.ds(base + (bat + 1) * nb * CH, nb * CH)],
                    m_v.at[1 - p], sems[1 - p])
            for j in range(nb):
                pltpu.sync_copy(m_v.at[p, pl.ds(j * CH, CH)],
                                acc_sh.at[idx_v.at[bat * nb + j]], add=True)
        plsc.subcore_barrier()
        pltpu.sync_copy(acc_sh.at[pl.ds(sid * stripe, stripe)],
                        out_hbm.at[cid, pl.ds(sid * stripe, stripe)])

    return k(msg, dst3, zeros_n)


def _msg_call(half, ea, xgp, stats, W1, b1, W2, b2, W3T, b3):
    off = half * HB
    grid_n = min(GRID - off, HB)
    return pl.pallas_call(
        _make_msg_body(off),
        grid=(grid_n,),
        in_specs=[
            pl.BlockSpec((B, ED), lambda b: (b + off, 0)),
            pl.BlockSpec((SB, G * IN), lambda b: (b, 0)),
            pl.BlockSpec((8, IN), lambda b: (0, 0)),
            pl.BlockSpec((ED, HID), lambda b: (0, 0)),
            pl.BlockSpec((1, HID), lambda b: (0, 0)),
            pl.BlockSpec((HID, HID), lambda b: (0, 0)),
            pl.BlockSpec((1, HID), lambda b: (0, 0)),
            pl.BlockSpec((IN * HID, HID), lambda b: (0, 0)),
            pl.BlockSpec((IN * HID, 1), lambda b: (0, 0)),
        ],
        out_specs=pl.BlockSpec((SB, G * HID), lambda b: (b, 0)),
        out_shape=jax.ShapeDtypeStruct((HE // G, G * HID), _f32),
    )(ea, xgp, stats, W1, b1, W2, b2, W3T, b3)


def kernel(x, edge_index, edge_attr, W1, b1, W2, b2, W3, b3, root, ncb,
           L1, bL1, L2, bL2):
    pad = E_PAD - E
    src_scan = _to_scan_order(jnp.pad(edge_index[0], (0, pad)))
    dst_scan = _to_scan_order(jnp.pad(edge_index[1], (0, pad),
                                      constant_values=jnp.int32(N)))
    src3 = src_scan.reshape(NH, NW, CPW_H, CH)
    dst3 = dst_scan.reshape(NH, NW, CPW_H, CH)
    zeros_n = jnp.zeros((N, HID), _f32)
    b1r, b2r = b1.reshape(1, HID), b2.reshape(1, HID)
    W3T = W3.T
    b3c = b3.reshape(IN * HID, 1)

    stats, r = pl.pallas_call(
        _stats_root_body,
        out_shape=[jax.ShapeDtypeStruct((8, IN), _f32),
                   jax.ShapeDtypeStruct((N, HID), _f32)],
    )(x, root, ncb.reshape(1, HID))

    xg0 = _sc_gather(x, src3[0]).reshape(HE // G, G * IN)
    msg0 = _msg_call(0, edge_attr, xg0, stats, W1, b1r, W2, b2r, W3T, b3c)
    p0 = _sc_scatter(msg0.reshape(HE, HID), dst3[0], zeros_n)

    out = pl.pallas_call(
        _final_body,
        out_shape=jax.ShapeDtypeStruct((N, OUT), _f32),
    )(p0, r, L1, bL1.reshape(1, HID), L2, bL2.reshape(1, OUT))
    return out


# speedup vs baseline: 4.6299x; 1.0028x over previous
import functools

import jax
import jax.numpy as jnp
from jax import lax
from jax.experimental import pallas as pl
from jax.experimental.pallas import tpu as pltpu
from jax.experimental.pallas import tpu_sc as plsc

N = 10000
E = 160000
IN = 32
OUT = 32
HID = 32
ED = 16
EPS = 1e-5

NW = 32
CH = 128
E_PAD = 163840
CPW = E_PAD // (NW * CH)
B = 16384
G = 4
SB = B // G
GRID = (E + B - 1) // B
P_ROWS = E_PAD // G
N_ACC = N + 8

NH = 1
HB = E_PAD // B // NH
HE = E_PAD // NH
CPW_H = CPW // NH

_f32 = jnp.float32


def _to_scan_order(v):
    return v.reshape(E_PAD // B, G, SB).transpose(0, 2, 1).reshape(-1)


def _stats_root_body(x_ref, root_ref, ncb_ref, stats_ref, r_ref):
    x = x_ref[...]
    mean = jnp.mean(x, axis=0, keepdims=True)
    xc = x - mean
    var = jnp.mean(xc * xc, axis=0, keepdims=True)
    istd = lax.rsqrt(var + EPS)
    xn = xc * istd
    r_ref[...] = jnp.dot(xn, root_ref[...], preferred_element_type=_f32) + ncb_ref[...]
    stats_ref[...] = jnp.concatenate(
        [mean, istd, jnp.zeros((6, IN), _f32)], axis=0)


def _make_msg_body(off):
    def body(ea_ref, xgp_ref, stats_ref, w1_ref, b1_ref, w2_ref, b2_ref,
             w3t_ref, b3_ref, out_ref):
        b = pl.program_id(0) + off
        ea = ea_ref[...]
        h1 = jnp.maximum(jnp.dot(ea, w1_ref[...], preferred_element_type=_f32)
                         + b1_ref[...], 0.0)
        h2 = jnp.maximum(jnp.dot(h1, w2_ref[...], preferred_element_type=_f32)
                         + b2_ref[...], 0.0)
        h2T = h2.T

        stats = stats_ref[...]
        meanr = jnp.concatenate([stats[0:1, :]] * G, axis=1)
        istdr = jnp.concatenate([stats[1:2, :]] * G, axis=1)
        xgn = (xgp_ref[...] - meanr) * istdr
        xgn3 = xgn.T.reshape(G, IN, SB)

        accs = []
        for g in range(G):
            wTg = jnp.maximum(
                jnp.dot(w3t_ref[...], h2T[:, g * SB:(g + 1) * SB],
                        preferred_element_type=_f32) + b3_ref[...], 0.0)
            acc = jnp.zeros((HID, SB), _f32)
            for i in range(IN):
                row = jnp.broadcast_to(xgn3[g, i:i + 1, :], (HID, SB))
                acc = acc + wTg[HID * i:HID * (i + 1), :] * row
            accs.append(acc)
        msgT = jnp.concatenate(accs, axis=0)

        rid = lax.broadcasted_iota(jnp.int32, (G * HID, SB), 0) // HID
        cid = lax.broadcasted_iota(jnp.int32, (G * HID, SB), 1)
        eid = b * B + rid * SB + cid
        msgT = jnp.where(eid < E, msgT, 0.0)
        out_ref[...] = msgT.T
    return body


def _final_body(p0_ref, r_ref, l1_ref, bl1_ref, l2_ref, bl2_ref, out_ref):
    agg = p0_ref[0] + p0_ref[1]
    o = jnp.maximum(agg + r_ref[...], 0.0)
    h = jnp.maximum(jnp.dot(o, l1_ref[...], preferred_element_type=_f32)
                    + bl1_ref[...], 0.0)
    out_ref[...] = jnp.dot(h, l2_ref[...], preferred_element_type=_f32) + bl2_ref[...]


def _sc_gather(x, src3):
    mesh = plsc.VectorSubcoreMesh(core_axis_name="c", subcore_axis_name="s")

    nb = 8
    nbat = CPW_H // nb

    @functools.partial(
        pl.kernel, mesh=mesh,
        out_type=jax.ShapeDtypeStruct((HE, IN), _f32),
        scratch_types=[
            pltpu.VMEM((CPW_H, CH), jnp.int32),
            pltpu.VMEM((2, nb * CH, IN), _f32),
            pltpu.SemaphoreType.DMA,
            pltpu.SemaphoreType.DMA,
            pltpu.SemaphoreType.DMA,
        ],
        compiler_params=pltpu.CompilerParams(use_tc_tiling_on_sc=False),
    )
    def k(x_hbm, src_hbm, out_hbm, idx_v, rows_v, sem_g, sem_w0, sem_w1):
        wid = lax.axis_index("s") * 2 + lax.axis_index("c")
        pltpu.sync_copy(src_hbm.at[wid], idx_v)
        sem_w = (sem_w0, sem_w1)
        hw = [None, None]
        for bat in range(nbat):
            p = bat % 2
            if hw[p] is not None:
                hw[p].wait()
            hg = [
                pltpu.async_copy(x_hbm.at[idx_v.at[bat * nb + j]],
                                 rows_v.at[p, pl.ds(j * CH, CH)], sem_g)
                for j in range(nb)
            ]
            for h in hg:
                h.wait()
            hw[p] = pltpu.async_copy(
                rows_v.at[p],
                out_hbm.at[pl.ds(wid * CPW_H * CH + bat * nb * CH, nb * CH)],
                sem_w[p])
        hw[0].wait()
        hw[1].wait()

    return k(x, src3)


def _sc_scatter(msg, dst3, zeros_n):
    mesh = plsc.VectorSubcoreMesh(core_axis_name="c", subcore_axis_name="s")
    stripe = N // 16

    nb = 8
    nbat = CPW_H // nb

    @functools.partial(
        pl.kernel, mesh=mesh,
        out_type=jax.ShapeDtypeStruct((2, N, HID), _f32),
        scratch_types=[
            pltpu.VMEM_SHARED((N_ACC, HID), _f32),
            pltpu.VMEM((CPW_H, CH), jnp.int32),
            pltpu.VMEM((2, nb * CH, HID), _f32),
            pltpu.SemaphoreType.DMA,
            pltpu.SemaphoreType.DMA,
        ],
        compiler_params=pltpu.CompilerParams(use_tc_tiling_on_sc=False),
    )
    def k(msg_hbm, dst_hbm, zeros_hbm, out_hbm, acc_sh, idx_v, m_v, sem0, sem1):
        cid = lax.axis_index("c")
        sid = lax.axis_index("s")
        wid = sid * 2 + cid
        base = wid * CPW_H * CH
        pltpu.sync_copy(zeros_hbm.at[pl.ds(sid * stripe, stripe)],
                        acc_sh.at[pl.ds(sid * stripe, stripe)])
        pltpu.sync_copy(dst_hbm.at[wid], idx_v)
        plsc.subcore_barrier()
        sems = (sem0, sem1)
        hl = pltpu.async_copy(msg_hbm.at[pl.ds(base, nb * CH)],
                              m_v.at[0], sems[0])
        for bat in range(nbat):
            p = bat % 2
            hl.wait()
            if bat + 1 < nbat:
                hl = pltpu.async_copy(
                    msg_hbm.at[pl.ds(base + (bat + 1) * nb * CH, nb * CH)],
                    m_v.at[1 - p], sems[1 - p])
            for j in range(nb):
                pltpu.sync_copy(m_v.at[p, pl.ds(j * CH, CH)],
                                acc_sh.at[idx_v.at[bat * nb + j]], add=True)
        plsc.subcore_barrier()
        pltpu.sync_copy(acc_sh.at[pl.ds(sid * stripe, stripe)],
                        out_hbm.at[cid, pl.ds(sid * stripe, stripe)])

    return k(msg, dst3, zeros_n)


def _msg_call(half, ea, xgp, stats, W1, b1, W2, b2, W3T, b3):
    off = half * HB
    grid_n = min(GRID - off, HB)
    return pl.pallas_call(
        _make_msg_body(off),
        grid=(grid_n,),
        in_specs=[
            pl.BlockSpec((B, ED), lambda b: (b + off, 0)),
            pl.BlockSpec((SB, G * IN), lambda b: (b, 0)),
            pl.BlockSpec((8, IN), lambda b: (0, 0)),
            pl.BlockSpec((ED, HID), lambda b: (0, 0)),
            pl.BlockSpec((1, HID), lambda b: (0, 0)),
            pl.BlockSpec((HID, HID), lambda b: (0, 0)),
            pl.BlockSpec((1, HID), lambda b: (0, 0)),
            pl.BlockSpec((IN * HID, HID), lambda b: (0, 0)),
            pl.BlockSpec((IN * HID, 1), lambda b: (0, 0)),
        ],
        out_specs=pl.BlockSpec((SB, G * HID), lambda b: (b, 0)),
        out_shape=jax.ShapeDtypeStruct((HE // G, G * HID), _f32),
    )(ea, xgp, stats, W1, b1, W2, b2, W3T, b3)


def kernel(x, edge_index, edge_attr, W1, b1, W2, b2, W3, b3, root, ncb,
           L1, bL1, L2, bL2):
    pad = E_PAD - E
    src_scan = _to_scan_order(jnp.pad(edge_index[0], (0, pad)))
    dst_scan = _to_scan_order(jnp.pad(edge_index[1], (0, pad),
                                      constant_values=jnp.int32(N)))
    src3 = src_scan.reshape(NH, NW, CPW_H, CH)
    dst3 = dst_scan.reshape(NH, NW, CPW_H, CH)
    zeros_n = jnp.zeros((N, HID), _f32)
    b1r, b2r = b1.reshape(1, HID), b2.reshape(1, HID)
    W3T = W3.T
    b3c = b3.reshape(IN * HID, 1)

    stats, r = pl.pallas_call(
        _stats_root_body,
        out_shape=[jax.ShapeDtypeStruct((8, IN), _f32),
                   jax.ShapeDtypeStruct((N, HID), _f32)],
    )(x, root, ncb.reshape(1, HID))

    xg0 = _sc_gather(x, src3[0]).reshape(HE // G, G * IN)
    msg0 = _msg_call(0, edge_attr, xg0, stats, W1, b1r, W2, b2r, W3T, b3c)
    p0 = _sc_scatter(msg0.reshape(HE, HID), dst3[0], zeros_n)

    out = pl.pallas_call(
        _final_body,
        out_shape=jax.ShapeDtypeStruct((N, OUT), _f32),
    )(p0, r, L1, bL1.reshape(1, HID), L2, bL2.reshape(1, OUT))
    return out
